# baseline (speedup 1.0000x reference)
import jax
import jax.numpy as jnp
from jax.experimental import pallas as pl
from jax.experimental.pallas import tpu as pltpu

_C, _HW = 96, 256
_HID, _E = 256, 12
_CB = 6
_KB = _CB * 64
_NB = 1024


def _body(thr_ref, x_ref, w1_ref, b1_ref, gam_ref, bet_ref,
          w2_ref, b2_ref, out_ref, acc_ref):
    s = pl.program_id(0)
    ns = pl.num_programs(0)

    @pl.when(s == 0)
    def _init():
        acc_ref[...] = jnp.zeros_like(acc_ref)

    x = x_ref[...]
    nb = x.shape[-1]
    x4 = x.reshape(_CB * 4, 64, nb)
    hs = x4[:, 0:16] + x4[:, 16:32] + x4[:, 32:48] + x4[:, 48:64]
    ss = [hs[:, 4 * j] + hs[:, 4 * j + 1] + hs[:, 4 * j + 2] + hs[:, 4 * j + 3]
          for j in range(4)]
    z = jnp.concatenate(
        [sj[:, None, :] for sj in ss for _ in range(4)], axis=1)
    z = z.reshape(_KB, nb)
    acc_ref[...] += jnp.dot(w1_ref[...], z,
                            preferred_element_type=jnp.float32)

    @pl.when(s == ns - 1)
    def _epilogue():
        x0 = acc_ref[...] + b1_ref[...]
        mu = jnp.mean(x0, axis=0, keepdims=True)
        var = jnp.mean((x0 - mu) ** 2, axis=0, keepdims=True)
        xn = (x0 - mu) / jnp.sqrt(var + 1e-5) * gam_ref[...] + bet_ref[...]
        xg = 0.5 * xn * (1.0 + jax.lax.erf(xn * 0.7071067811865476))
        lg = jnp.dot(w2_ref[...], xg,
                     preferred_element_type=jnp.float32) + b2_ref[...]

        row = jax.lax.broadcasted_iota(jnp.int32, lg.shape, 0)
        live = row < _E
        m = jnp.max(jnp.where(live, lg, -jnp.inf), axis=0, keepdims=True)
        e = jnp.where(live, jnp.exp(lg - m), 0.0)
        p = e / jnp.sum(e, axis=0, keepdims=True)

        thr = thr_ref[0]
        cnt = jnp.sum((p > thr).astype(jnp.float32), axis=0, keepdims=True)
        ak = jnp.clip(cnt, 1.0, 4.0)
        ak_i = ak.astype(jnp.int32)

        rank = jnp.zeros(lg.shape, jnp.int32)
        for j in range(_E):
            pj = p[j:j + 1, :]
            rank += (pj > p).astype(jnp.int32)
            rank += ((pj == p) & (row > j)).astype(jnp.int32)
        g = jnp.where(rank < ak_i, p, 0.0)
        gs = jnp.sum(g, axis=0, keepdims=True)
        out_ref[...] = g / (gs + 1e-8) + jnp.where(row == _E, ak, 0.0)


def kernel(feat, W1, b1, gamma, beta, W2, b2, log_threshold):
    b = feat.shape[0]
    featT = feat.transpose(1, 2, 3, 0).reshape(_C, _HW, b)

    w1r = W1.reshape(_HID, _C, 4, 4)
    w1e = jnp.broadcast_to(w1r[:, :, :, :, None],
                           (_HID, _C, 4, 4, 4)).reshape(_HID, _C * 64)
    w1e = w1e * (1.0 / 64.0)

    w2p = jnp.zeros((16, _HID), W2.dtype).at[:_E].set(W2)
    b2p = jnp.zeros((16, 1), b2.dtype).at[:_E, 0].set(b2)
    thr = jnp.maximum(jax.nn.sigmoid(log_threshold), 1.0 / _E).reshape(1)

    ns = _C // _CB
    outT = pl.pallas_call(
        _body,
        grid=(ns,),
        in_specs=[
            pl.BlockSpec(memory_space=pltpu.SMEM),
            pl.BlockSpec((_CB, _HW, b), lambda s: (s, 0, 0)),
            pl.BlockSpec((_HID, _KB), lambda s: (0, s)),
            pl.BlockSpec((_HID, 1), lambda s: (0, 0)),
            pl.BlockSpec((_HID, 1), lambda s: (0, 0)),
            pl.BlockSpec((_HID, 1), lambda s: (0, 0)),
            pl.BlockSpec((16, _HID), lambda s: (0, 0)),
            pl.BlockSpec((16, 1), lambda s: (0, 0)),
        ],
        out_specs=pl.BlockSpec((16, b), lambda s: (0, 0)),
        out_shape=jax.ShapeDtypeStruct((16, b), jnp.float32),
        scratch_shapes=[pltpu.VMEM((_HID, b), jnp.float32)],
        compiler_params=pltpu.CompilerParams(
            dimension_semantics=("arbitrary",)),
    )(thr, featT, w1e, b1.reshape(_HID, 1), gamma.reshape(_HID, 1),
      beta.reshape(_HID, 1), w2p, b2p)

    t = outT.T
    gate = t[:, :_E]
    ak = t[:, _E]
    balance_loss = jnp.asarray(0.0, dtype=jnp.float32)
    return (gate, ak, balance_loss)


# speedup vs baseline: 1.0938x; 1.0938x over previous
import jax
import jax.numpy as jnp
from jax.experimental import pallas as pl
from jax.experimental.pallas import tpu as pltpu

_C, _HW = 96, 256
_HID, _E = 256, 12
_CB = 4
_KB = _CB * 16
_NB = 1024


def _body(thr_ref, x_ref, w1_ref, b1_ref, gam_ref, bet_ref,
          w2_ref, b2_ref, out_ref, acc_ref):
    s = pl.program_id(0)
    ns = pl.num_programs(0)

    @pl.when(s == 0)
    def _init():
        acc_ref[...] = jnp.zeros_like(acc_ref)

    x = x_ref[...]
    nb = x.shape[-1]
    x4 = x.reshape(_CB * 4, 64, nb)
    hs = x4[:, 0:16] + x4[:, 16:32] + x4[:, 32:48] + x4[:, 48:64]
    ss = [hs[:, 4 * j] + hs[:, 4 * j + 1] + hs[:, 4 * j + 2] + hs[:, 4 * j + 3]
          for j in range(4)]
    z = jnp.concatenate(ss, axis=0)
    w = w1_ref[...].reshape(_HID, _KB) * (1.0 / 16.0)
    acc_ref[...] += jnp.dot(w, z, preferred_element_type=jnp.float32)

    @pl.when(s == ns - 1)
    def _epilogue():
        x0 = acc_ref[...] + b1_ref[...]
        mu = jnp.mean(x0, axis=0, keepdims=True)
        var = jnp.mean((x0 - mu) ** 2, axis=0, keepdims=True)
        xn = (x0 - mu) / jnp.sqrt(var + 1e-5) * gam_ref[...] + bet_ref[...]
        xg = 0.5 * xn * (1.0 + jax.lax.erf(xn * 0.7071067811865476))
        lg = jnp.dot(w2_ref[...], xg,
                     preferred_element_type=jnp.float32) + b2_ref[...]

        row = jax.lax.broadcasted_iota(jnp.int32, lg.shape, 0)
        live = row < _E
        m = jnp.max(jnp.where(live, lg, -jnp.inf), axis=0, keepdims=True)
        e = jnp.where(live, jnp.exp(lg - m), 0.0)
        p = e / jnp.sum(e, axis=0, keepdims=True)

        thr = thr_ref[0]
        cnt = jnp.sum((p > thr).astype(jnp.float32), axis=0, keepdims=True)
        ak = jnp.clip(cnt, 1.0, 4.0)
        ak_i = ak.astype(jnp.int32)

        rank = jnp.zeros(lg.shape, jnp.int32)
        for j in range(_E):
            pj = p[j:j + 1, :]
            rank += (pj > p).astype(jnp.int32)
            rank += ((pj == p) & (row > j)).astype(jnp.int32)
        g = jnp.where(rank < ak_i, p, 0.0)
        gs = jnp.sum(g, axis=0, keepdims=True)
        out_ref[...] = g / (gs + 1e-8) + jnp.where(row == _E, ak, 0.0)


def kernel(feat, W1, b1, gamma, beta, W2, b2, log_threshold):
    b = feat.shape[0]
    featT = feat.transpose(1, 2, 3, 0).reshape(_C, _HW, b)

    ns = _C // _CB
    w1b = (W1.reshape(_HID, ns, _CB, 4, 4)
           .transpose(1, 0, 4, 2, 3).reshape(ns, _HID, _KB))
    w2p = jnp.zeros((16, _HID), W2.dtype).at[:_E].set(W2)
    b2p = jnp.zeros((16, 1), b2.dtype).at[:_E, 0].set(b2)
    thr = jnp.maximum(jax.nn.sigmoid(log_threshold), 1.0 / _E).reshape(1)

    outT = pl.pallas_call(
        _body,
        grid=(ns,),
        in_specs=[
            pl.BlockSpec(memory_space=pltpu.SMEM),
            pl.BlockSpec((_CB, _HW, b), lambda s: (s, 0, 0)),
            pl.BlockSpec((1, _HID, _KB), lambda s: (s, 0, 0)),
            pl.BlockSpec((_HID, 1), lambda s: (0, 0)),
            pl.BlockSpec((_HID, 1), lambda s: (0, 0)),
            pl.BlockSpec((_HID, 1), lambda s: (0, 0)),
            pl.BlockSpec((16, _HID), lambda s: (0, 0)),
            pl.BlockSpec((16, 1), lambda s: (0, 0)),
        ],
        out_specs=pl.BlockSpec((16, b), lambda s: (0, 0)),
        out_shape=jax.ShapeDtypeStruct((16, b), jnp.float32),
        scratch_shapes=[pltpu.VMEM((_HID, b), jnp.float32)],
        compiler_params=pltpu.CompilerParams(
            dimension_semantics=("arbitrary",)),
    )(thr, featT, w1b, b1.reshape(_HID, 1), gamma.reshape(_HID, 1),
      beta.reshape(_HID, 1), w2p, b2p)

    t = outT.T
    gate = t[:, :_E]
    ak = t[:, _E]
    balance_loss = jnp.asarray(0.0, dtype=jnp.float32)
    return (gate, ak, balance_loss)
